# Initial kernel scaffold; baseline (speedup 1.0000x reference)
import jax
import jax.numpy as jnp
from jax.experimental import pallas as pl


def kernel(hidden_states, pos):
    raise NotImplementedError("write your pallas kernel here")


# speedup vs baseline: 1.2618x; 1.2618x over previous
import functools

import jax
import jax.numpy as jnp
from jax import lax
from jax.experimental import pallas as pl
from jax.experimental.pallas import tpu as pltpu
from jax.experimental.pallas import tpu_sc as plsc

_BATCH = 16
_SEQ = 2048
_MAX_SENT = 128
_DIM = 1024

_NUM_CORES = 2
_NUM_SUBCORES = 16
_NUM_WORKERS = _NUM_CORES * _NUM_SUBCORES
_ROWS = _BATCH * _MAX_SENT
_ROWS_PER_WORKER = _ROWS // _NUM_WORKERS
_LANES = 16


def _gather_body(hs_hbm, pos_hbm, out_hbm, idx_v, rows_v, sem):
    wid = lax.axis_index("s") * _NUM_CORES + lax.axis_index("c")
    base = wid * _ROWS_PER_WORKER
    pltpu.sync_copy(pos_hbm.at[pl.ds(base, _ROWS_PER_WORKER)], idx_v)
    offset = (base // _MAX_SENT) * _SEQ
    for r in range(_ROWS_PER_WORKER // _LANES):
        sl = pl.ds(r * _LANES, _LANES)
        idx_v[sl] = idx_v[sl] + offset
    pltpu.async_copy(hs_hbm.at[idx_v], rows_v, sem).wait()
    pltpu.sync_copy(rows_v, out_hbm.at[pl.ds(base, _ROWS_PER_WORKER)])


_gather = functools.partial(
    pl.kernel,
    out_type=jax.ShapeDtypeStruct((_ROWS, _DIM), jnp.float32),
    mesh=plsc.VectorSubcoreMesh(core_axis_name="c", subcore_axis_name="s"),
    scratch_types=[
        pltpu.VMEM((_ROWS_PER_WORKER,), jnp.int32),
        pltpu.VMEM((_ROWS_PER_WORKER, _DIM), jnp.float32),
        pltpu.SemaphoreType.DMA,
    ],
)(_gather_body)


@jax.jit
def kernel(hidden_states, pos):
    hs_flat = hidden_states.reshape(_BATCH * _SEQ, _DIM)
    pos_flat = pos.reshape(_ROWS)
    out = _gather(hs_flat, pos_flat)
    return out.reshape(_BATCH, _MAX_SENT, _DIM)
